# baseline (speedup 1.0000x reference)
import functools

import jax
import jax.numpy as jnp
from jax import lax
from jax.experimental import pallas as pl
from jax.experimental.pallas import tpu as pltpu
from jax.experimental.pallas import tpu_sc as plsc

_SPAN = 64
_B = 16
_S = 2048
_L = 16
_NV = _S // _L


def _bcast_last(x):
    idx = jnp.full((_L,), _L - 1, dtype=jnp.int32)
    return x.at[idx].get(mode="promise_in_bounds")


_mesh = plsc.VectorSubcoreMesh(core_axis_name="c", subcore_axis_name="s")


@functools.partial(
    pl.kernel,
    mesh=_mesh,
    out_type=jax.ShapeDtypeStruct((_B, _S), jnp.int32),
    compiler_params=pltpu.CompilerParams(needs_layout_passes=False),
    scratch_types=[
        pltpu.VMEM((_S,), jnp.int32),
        pltpu.VMEM((_S,), jnp.int32),
    ],
)
def _seg_kernel(mask_hbm, out_hbm, m_buf, seg_buf):
    nc = 2
    w = lax.axis_index("s") * nc + lax.axis_index("c")

    @pl.when(w < _B)
    def _():
        pltpu.sync_copy(mask_hbm.at[w], m_buf)
        lane = lax.iota(jnp.int32, _L)
        shift_idx = jnp.maximum(lane - 1, 0)

        def body(i, carries):
            prev_c, rs_c, cid_c, lc_c = carries
            cur = m_buf[pl.ds(i * _L, _L)]
            t = lane + i * _L
            sh = cur.at[shift_idx].get(mode="promise_in_bounds")
            prv = jnp.where(lane == 0, prev_c, sh)
            change = (cur == 1) != (prv == 1)
            boundary = change | (t == 0)
            bidx = jnp.where(boundary, t, jnp.int32(-1))
            rs = jnp.maximum(plsc.cummax(bidx), rs_c)
            cb = ((t - rs) & (_SPAN - 1)) == 0
            cid = plsc.cumsum(jnp.where(cb, jnp.int32(1), jnp.int32(0))) + cid_c
            lc = jnp.maximum(lc_c, jnp.where(change, t, jnp.int32(0)))
            seg_buf[pl.ds(i * _L, _L)] = cid
            return (_bcast_last(cur), _bcast_last(rs), _bcast_last(cid), lc)

        zeros = jnp.zeros((_L,), jnp.int32)
        neg1 = jnp.full((_L,), -1, jnp.int32)
        _, _, _, lc_f = lax.fori_loop(0, _NV, body, (zeros, neg1, neg1, zeros))

        last_vreg = m_buf[pl.ds(_S - _L, _L)]
        keep_last = jnp.max(jnp.where(lane == _L - 1,
                                      jnp.where(last_vreg == 1, jnp.int32(1), jnp.int32(0)),
                                      jnp.int32(0)))
        last_change = jnp.max(lc_f)
        extent = jnp.where(keep_last == 1, jnp.int32(_S), last_change + 1)

        def fix_body(i, _):
            t = lane + i * _L
            v = seg_buf[pl.ds(i * _L, _L)]
            seg_buf[pl.ds(i * _L, _L)] = jnp.where(t < extent, v, jnp.int32(-1))
            return _

        lax.fori_loop(extent // _L, _NV, fix_body, jnp.int32(0))

        pltpu.sync_copy(seg_buf, out_hbm.at[w])


def kernel(inp, padding_mask, regular_tokens_mask):
    del inp, padding_mask
    return _seg_kernel(regular_tokens_mask)
